# baseline (speedup 1.0000x reference)
import functools

import jax
import jax.numpy as jnp
from jax import lax
from jax.experimental import pallas as pl
from jax.experimental.pallas import tpu as pltpu
from jax.experimental.pallas import tpu_sc as plsc

FEAT = 128
N_NODES = 10000
N_EDGES = 320000
NW = 32
E_PER_W = N_EDGES // NW
CHUNK = 80
N_CHUNKS = E_PER_W // CHUNK
N_PAIRS = (N_CHUNKS - 1) // 2
LANES = 16
KSLICE = FEAT // LANES
KBLK = FEAT // (2 * LANES)


def _precompute_tables(x, w1a, w1b, b1r):
    rows = 2000
    grid = x.shape[0] // rows

    def body(x_ref, wa_ref, wb_ref, b1_ref, a_ref, b_ref):
        xb = x_ref[...]
        a_ref[...] = (
            jnp.dot(xb, wa_ref[...], preferred_element_type=jnp.float32)
            + b1_ref[...]
        ).astype(jnp.bfloat16)
        b_ref[...] = jnp.dot(
            xb, wb_ref[...], preferred_element_type=jnp.float32
        ).astype(jnp.bfloat16)

    def _pack_i32(t):
        return lax.bitcast_convert_type(
            t.reshape(t.shape[0], FEAT // 2, 2), jnp.int32
        )

    a_tab, b_tab = pl.pallas_call(
        body,
        grid=(grid,),
        in_specs=[
            pl.BlockSpec((rows, FEAT), lambda i: (i, 0)),
            pl.BlockSpec((FEAT, FEAT), lambda i: (0, 0)),
            pl.BlockSpec((FEAT, FEAT), lambda i: (0, 0)),
            pl.BlockSpec((1, FEAT), lambda i: (0, 0)),
        ],
        out_specs=[
            pl.BlockSpec((rows, FEAT), lambda i: (i, 0)),
            pl.BlockSpec((rows, FEAT), lambda i: (i, 0)),
        ],
        out_shape=[
            jax.ShapeDtypeStruct((x.shape[0], FEAT), jnp.bfloat16),
            jax.ShapeDtypeStruct((x.shape[0], FEAT), jnp.bfloat16),
        ],
    )(x, w1a, w1b, b1r)
    return _pack_i32(a_tab), _pack_i32(b_tab)


def _edge_scores(a_tab, b_tab, src, dst, w2r, b2v):
    mesh = plsc.VectorSubcoreMesh(core_axis_name="c", subcore_axis_name="s")

    @functools.partial(
        pl.kernel,
        mesh=mesh,
        out_type=jax.ShapeDtypeStruct((N_EDGES,), jnp.float32),
        scratch_types=[
            pltpu.VMEM((E_PER_W,), jnp.int32),
            pltpu.VMEM((E_PER_W,), jnp.int32),
            pltpu.VMEM((2, CHUNK, FEAT // 2), jnp.int32),
            pltpu.VMEM((2, CHUNK, FEAT // 2), jnp.int32),
            pltpu.VMEM((2, CHUNK), jnp.float32),
            pltpu.VMEM((KBLK, 2 * LANES), jnp.bfloat16),
            pltpu.VMEM((LANES,), jnp.float32),
            pltpu.SemaphoreType.DMA,
            (pltpu.SemaphoreType.DMA,) * 2,
            (pltpu.SemaphoreType.DMA,) * 2,
            (pltpu.SemaphoreType.DMA,) * 2,
        ],
        compiler_params=pltpu.CompilerParams(
            needs_layout_passes=False, use_tc_tiling_on_sc=False
        ),
    )
    def k(a_hbm, b_hbm, src_hbm, dst_hbm, w2_hbm, b2_hbm, out_hbm,
          sidx, didx, arows, brows, outv, w2v, b2vv,
          sem_i, sems_a, sems_b, sems_o):
        sid = lax.axis_index("s")
        wid = sid * 2 + lax.axis_index("c")
        base0 = wid * E_PER_W
        ci1 = pltpu.async_copy(src_hbm.at[pl.ds(base0, E_PER_W)], sidx, sem_i)
        ci2 = pltpu.async_copy(dst_hbm.at[pl.ds(base0, E_PER_W)], didx, sem_i)
        pltpu.sync_copy(w2_hbm, w2v)
        pltpu.sync_copy(b2_hbm, b2vv)
        ci1.wait()
        ci2.wait()
        b2vec = b2vv[...]
        w2k = [w2v[kk, :] for kk in range(KBLK)]
        lane_iota = lax.broadcasted_iota(jnp.int32, (LANES,), 0)
        last_mask = lane_iota == (LANES - 1)
        rot_idx = [(lane_iota + r) & (LANES - 1) for r in (8, 4, 2, 1)]

        def issue(c, buf):
            off = c * CHUNK
            pltpu.async_copy(
                a_hbm.at[sidx.at[pl.ds(off, CHUNK)]], arows.at[buf], sems_a[buf]
            )
            pltpu.async_copy(
                b_hbm.at[didx.at[pl.ds(off, CHUNK)]], brows.at[buf], sems_b[buf]
            )

        def wait_rows(buf):
            pltpu.make_async_copy(
                a_hbm.at[sidx.at[pl.ds(0, CHUNK)]], arows.at[buf], sems_a[buf]
            ).wait()
            pltpu.make_async_copy(
                b_hbm.at[didx.at[pl.ds(0, CHUNK)]], brows.at[buf], sems_b[buf]
            ).wait()

        def drain_out(buf):
            pltpu.make_async_copy(
                outv.at[buf], out_hbm.at[pl.ds(0, CHUNK)], sems_o[buf]
            ).wait()

        def compute(c, buf):
            ar = arows.at[buf]
            br = brows.at[buf]
            ov = outv.at[buf]

            @pl.when(c >= 2)
            def _():
                drain_out(buf)

            @plsc.parallel_loop(0, CHUNK, unroll=4)
            def edge_body(e):
                prods = []
                for kk in range(KBLK):
                    sl = pl.ds(kk * LANES, LANES)
                    va = plsc.bitcast(ar[e, sl], jnp.bfloat16)
                    vb = plsc.bitcast(br[e, sl], jnp.bfloat16)
                    r = jnp.maximum(va + vb, jnp.bfloat16(0.0))
                    prods.append(r * w2k[kk])
                t = (prods[0] + prods[1]) + (prods[2] + prods[3])
                ti = plsc.bitcast(t, jnp.int32)
                lo = plsc.bitcast(ti << 16, jnp.float32)
                hi = plsc.bitcast(ti & jnp.int32(-65536), jnp.float32)
                tot = lo + hi
                for ridx in rot_idx:
                    tot = tot + tot[ridx]
                plsc.store_scatter(
                    ov,
                    [jnp.full((LANES,), 0, jnp.int32) + e],
                    tot,
                    mask=last_mask,
                )

            for j in range(CHUNK // LANES):
                sl = pl.ds(j * LANES, LANES)
                ov[sl] = 1.0 / (1.0 + jnp.exp(-(ov[sl] + b2vec)))
            pltpu.async_copy(
                ov, out_hbm.at[pl.ds(base0 + c * CHUNK, CHUNK)], sems_o[buf]
            )

        issue(0, 0)

        def pair_body(p, carry):
            c0 = 2 * p
            wait_rows(0)
            issue(c0 + 1, 1)
            compute(c0, 0)
            wait_rows(1)
            issue(c0 + 2, 0)
            compute(c0 + 1, 1)
            return carry

        lax.fori_loop(0, N_PAIRS, pair_body, 0)
        wait_rows(0)
        compute(jnp.int32(N_CHUNKS - 1), 0)
        drain_out(0)
        drain_out(1)

    return k(a_tab, b_tab, src, dst, w2r, b2v)


def kernel(x, edge_index, W1, b1, W2, b2):
    w1a = W1[:FEAT]
    w1b = W1[FEAT:]
    b1r = b1.reshape(1, FEAT)
    a_tab, b_tab = _precompute_tables(x, w1a, w1b, b1r)
    src = edge_index[0]
    dst = edge_index[1]
    w2r = W2.reshape(KBLK, 2 * LANES).astype(jnp.bfloat16)
    b2v = jnp.broadcast_to(b2, (LANES,))
    return _edge_scores(a_tab, b_tab, src, dst, w2r, b2v)


# speedup vs baseline: 1.3853x; 1.3853x over previous
import functools

import jax
import jax.numpy as jnp
from jax import lax
from jax.experimental import pallas as pl
from jax.experimental.pallas import tpu as pltpu
from jax.experimental.pallas import tpu_sc as plsc

FEAT = 128
N_NODES = 10000
N_EDGES = 320000
NW = 32
E_PER_W = N_EDGES // NW
CHUNK = 80
N_CHUNKS = E_PER_W // CHUNK
N_PAIRS = (N_CHUNKS - 1) // 2
LANES = 16
KSLICE = FEAT // LANES
KBLK = FEAT // (2 * LANES)


def _precompute_tables(x, w1a, w1b, b1r):
    rows = 2000
    grid = x.shape[0] // rows

    def _pack(t):
        tb = t.astype(jnp.bfloat16)
        lo = lax.bitcast_convert_type(tb[:, : FEAT // 2], jnp.uint16)
        hi = lax.bitcast_convert_type(tb[:, FEAT // 2 :], jnp.uint16)
        return lo.astype(jnp.int32) | (hi.astype(jnp.int32) << 16)

    def body(x_ref, wa_ref, wb_ref, b1_ref, a_ref, b_ref):
        xb = x_ref[...]
        a_ref[...] = _pack(
            jnp.dot(xb, wa_ref[...], preferred_element_type=jnp.float32)
            + b1_ref[...]
        )
        b_ref[...] = _pack(
            jnp.dot(xb, wb_ref[...], preferred_element_type=jnp.float32)
        )

    return pl.pallas_call(
        body,
        grid=(grid,),
        in_specs=[
            pl.BlockSpec((rows, FEAT), lambda i: (i, 0)),
            pl.BlockSpec((FEAT, FEAT), lambda i: (0, 0)),
            pl.BlockSpec((FEAT, FEAT), lambda i: (0, 0)),
            pl.BlockSpec((1, FEAT), lambda i: (0, 0)),
        ],
        out_specs=[
            pl.BlockSpec((rows, FEAT // 2), lambda i: (i, 0)),
            pl.BlockSpec((rows, FEAT // 2), lambda i: (i, 0)),
        ],
        out_shape=[
            jax.ShapeDtypeStruct((x.shape[0], FEAT // 2), jnp.int32),
            jax.ShapeDtypeStruct((x.shape[0], FEAT // 2), jnp.int32),
        ],
    )(x, w1a, w1b, b1r)


def _edge_scores(a_tab, b_tab, edge_index, w2r, b2v):
    mesh = plsc.VectorSubcoreMesh(core_axis_name="c", subcore_axis_name="s")

    @functools.partial(
        pl.kernel,
        mesh=mesh,
        out_type=jax.ShapeDtypeStruct((N_EDGES,), jnp.float32),
        scratch_types=[
            pltpu.VMEM((E_PER_W,), jnp.int32),
            pltpu.VMEM((E_PER_W,), jnp.int32),
            pltpu.VMEM((2, CHUNK, FEAT // 2), jnp.int32),
            pltpu.VMEM((2, CHUNK, FEAT // 2), jnp.int32),
            pltpu.VMEM((2, CHUNK), jnp.float32),
            pltpu.VMEM((KBLK, 2 * LANES), jnp.bfloat16),
            pltpu.VMEM((LANES,), jnp.float32),
            pltpu.SemaphoreType.DMA,
            (pltpu.SemaphoreType.DMA,) * 2,
            (pltpu.SemaphoreType.DMA,) * 2,
            (pltpu.SemaphoreType.DMA,) * 2,
        ],
        compiler_params=pltpu.CompilerParams(
            needs_layout_passes=False, use_tc_tiling_on_sc=False
        ),
    )
    def k(a_hbm, b_hbm, ei_hbm, w2_hbm, b2_hbm, out_hbm,
          sidx, didx, arows, brows, outv, w2v, b2vv,
          sem_i, sems_a, sems_b, sems_o):
        sid = lax.axis_index("s")
        wid = sid * 2 + lax.axis_index("c")
        base0 = wid * E_PER_W
        ci1 = pltpu.async_copy(ei_hbm.at[0, pl.ds(base0, E_PER_W)], sidx, sem_i)
        ci2 = pltpu.async_copy(ei_hbm.at[1, pl.ds(base0, E_PER_W)], didx, sem_i)
        pltpu.sync_copy(w2_hbm, w2v)
        pltpu.sync_copy(b2_hbm, b2vv)
        ci1.wait()
        ci2.wait()
        b2vec = b2vv[...]
        w2k = [w2v[kk, :] for kk in range(KBLK)]
        lane_iota = lax.broadcasted_iota(jnp.int32, (LANES,), 0)
        last_mask = lane_iota == (LANES - 1)
        rot_idx = [(lane_iota + r) & (LANES - 1) for r in (8, 4, 2, 1)]

        def issue(c, buf):
            off = c * CHUNK
            pltpu.async_copy(
                a_hbm.at[sidx.at[pl.ds(off, CHUNK)]], arows.at[buf], sems_a[buf]
            )
            pltpu.async_copy(
                b_hbm.at[didx.at[pl.ds(off, CHUNK)]], brows.at[buf], sems_b[buf]
            )

        def wait_rows(buf):
            pltpu.make_async_copy(
                a_hbm.at[sidx.at[pl.ds(0, CHUNK)]], arows.at[buf], sems_a[buf]
            ).wait()
            pltpu.make_async_copy(
                b_hbm.at[didx.at[pl.ds(0, CHUNK)]], brows.at[buf], sems_b[buf]
            ).wait()

        def drain_out(buf):
            pltpu.make_async_copy(
                outv.at[buf], out_hbm.at[pl.ds(0, CHUNK)], sems_o[buf]
            ).wait()

        def compute(c, buf):
            ar = arows.at[buf]
            br = brows.at[buf]
            ov = outv.at[buf]

            @pl.when(c >= 2)
            def _():
                drain_out(buf)

            @plsc.parallel_loop(0, CHUNK, unroll=4)
            def edge_body(e):
                prods = []
                for kk in range(KBLK):
                    sl = pl.ds(kk * LANES, LANES)
                    va = plsc.bitcast(ar[e, sl], jnp.bfloat16)
                    vb = plsc.bitcast(br[e, sl], jnp.bfloat16)
                    r = jnp.maximum(va + vb, jnp.bfloat16(0.0))
                    prods.append(r * w2k[kk])
                t = (prods[0] + prods[1]) + (prods[2] + prods[3])
                ti = plsc.bitcast(t, jnp.int32)
                lo = plsc.bitcast(ti << 16, jnp.float32)
                hi = plsc.bitcast(ti & jnp.int32(-65536), jnp.float32)
                tot = lo + hi
                for ridx in rot_idx:
                    tot = tot + tot[ridx]
                plsc.store_scatter(
                    ov,
                    [jnp.full((LANES,), 0, jnp.int32) + e],
                    tot,
                    mask=last_mask,
                )

            for j in range(CHUNK // LANES):
                sl = pl.ds(j * LANES, LANES)
                ov[sl] = 1.0 / (1.0 + jnp.exp(-(ov[sl] + b2vec)))
            pltpu.async_copy(
                ov, out_hbm.at[pl.ds(base0 + c * CHUNK, CHUNK)], sems_o[buf]
            )

        issue(0, 0)

        def pair_body(p, carry):
            c0 = 2 * p
            wait_rows(0)
            issue(c0 + 1, 1)
            compute(c0, 0)
            wait_rows(1)
            issue(c0 + 2, 0)
            compute(c0 + 1, 1)
            return carry

        lax.fori_loop(0, N_PAIRS, pair_body, 0)
        wait_rows(0)
        compute(jnp.int32(N_CHUNKS - 1), 0)
        drain_out(0)
        drain_out(1)

    return k(a_tab, b_tab, edge_index, w2r, b2v)


def kernel(x, edge_index, W1, b1, W2, b2):
    w1a = W1[:FEAT]
    w1b = W1[FEAT:]
    b1r = b1.reshape(1, FEAT)
    a_tab, b_tab = _precompute_tables(x, w1a, w1b, b1r)
    w2f = W2.reshape(FEAT)
    w2r = jnp.stack(
        [w2f[: FEAT // 2].reshape(KBLK, LANES), w2f[FEAT // 2 :].reshape(KBLK, LANES)],
        axis=-1,
    ).reshape(KBLK, 2 * LANES).astype(jnp.bfloat16)
    b2v = jnp.broadcast_to(b2, (LANES,))
    return _edge_scores(a_tab, b_tab, edge_index, w2r, b2v)


# speedup vs baseline: 1.9483x; 1.4064x over previous
import functools

import jax
import jax.numpy as jnp
from jax import lax
from jax.experimental import pallas as pl
from jax.experimental.pallas import tpu as pltpu
from jax.experimental.pallas import tpu_sc as plsc

FEAT = 128
N_NODES = 10000
N_EDGES = 320000
NW = 32
E_PER_W = N_EDGES // NW
CHUNK = 80
N_CHUNKS = E_PER_W // CHUNK
N_PAIRS = (N_CHUNKS - 1) // 2
LANES = 16
KSLICE = FEAT // LANES
KBLK = FEAT // (2 * LANES)


def _precompute_tables(x, w1a, w1b, b1r):
    rows = 2000
    grid = x.shape[0] // rows

    def _pack(t):
        tb = t.astype(jnp.bfloat16)
        lo = lax.bitcast_convert_type(tb[:, : FEAT // 2], jnp.uint16)
        hi = lax.bitcast_convert_type(tb[:, FEAT // 2 :], jnp.uint16)
        return lo.astype(jnp.int32) | (hi.astype(jnp.int32) << 16)

    def body(x_ref, wa_ref, wb_ref, b1_ref, a_ref, b_ref):
        xb = x_ref[...]
        a_ref[...] = _pack(
            jnp.dot(xb, wa_ref[...], preferred_element_type=jnp.float32)
            + b1_ref[...]
        )
        b_ref[...] = _pack(
            jnp.dot(xb, wb_ref[...], preferred_element_type=jnp.float32)
        )

    return pl.pallas_call(
        body,
        grid=(grid,),
        in_specs=[
            pl.BlockSpec((rows, FEAT), lambda i: (i, 0)),
            pl.BlockSpec((FEAT, FEAT), lambda i: (0, 0)),
            pl.BlockSpec((FEAT, FEAT), lambda i: (0, 0)),
            pl.BlockSpec((1, FEAT), lambda i: (0, 0)),
        ],
        out_specs=[
            pl.BlockSpec((rows, FEAT // 2), lambda i: (i, 0)),
            pl.BlockSpec((rows, FEAT // 2), lambda i: (i, 0)),
        ],
        out_shape=[
            jax.ShapeDtypeStruct((x.shape[0], FEAT // 2), jnp.int32),
            jax.ShapeDtypeStruct((x.shape[0], FEAT // 2), jnp.int32),
        ],
    )(x, w1a, w1b, b1r)


def _edge_scores(a_tab, b_tab, edge_index, w2r, b2v):
    mesh = plsc.VectorSubcoreMesh(core_axis_name="c", subcore_axis_name="s")

    @functools.partial(
        pl.kernel,
        mesh=mesh,
        out_type=jax.ShapeDtypeStruct((N_EDGES,), jnp.float32),
        scratch_types=[
            pltpu.VMEM((E_PER_W,), jnp.int32),
            pltpu.VMEM((E_PER_W,), jnp.int32),
            pltpu.VMEM((2, CHUNK, FEAT // 2), jnp.int32),
            pltpu.VMEM((2, CHUNK, FEAT // 2), jnp.int32),
            pltpu.VMEM_SHARED((N_NODES, FEAT // 2), jnp.int32),
            pltpu.VMEM_SHARED((N_NODES, FEAT // 2), jnp.int32),
            pltpu.VMEM((2, CHUNK), jnp.float32),
            pltpu.VMEM((KBLK, 2 * LANES), jnp.bfloat16),
            pltpu.VMEM((LANES,), jnp.float32),
            pltpu.SemaphoreType.DMA,
            (pltpu.SemaphoreType.DMA,) * 2,
            (pltpu.SemaphoreType.DMA,) * 2,
            (pltpu.SemaphoreType.DMA,) * 2,
        ],
        compiler_params=pltpu.CompilerParams(
            needs_layout_passes=False, use_tc_tiling_on_sc=False
        ),
    )
    def k(a_hbm, b_hbm, ei_hbm, w2_hbm, b2_hbm, out_hbm,
          sidx, didx, arows, brows, aspm, bspm, outv, w2v, b2vv,
          sem_i, sems_a, sems_b, sems_o):
        sid = lax.axis_index("s")
        wid = sid * 2 + lax.axis_index("c")
        base0 = wid * E_PER_W
        ci1 = pltpu.async_copy(ei_hbm.at[0, pl.ds(base0, E_PER_W)], sidx, sem_i)
        ci2 = pltpu.async_copy(ei_hbm.at[1, pl.ds(base0, E_PER_W)], didx, sem_i)
        rpt = N_NODES // 16
        stage = pl.ds(sid * rpt, rpt)
        pltpu.sync_copy(a_hbm.at[stage], aspm.at[stage])
        pltpu.sync_copy(b_hbm.at[stage], bspm.at[stage])
        pltpu.sync_copy(w2_hbm, w2v)
        pltpu.sync_copy(b2_hbm, b2vv)
        ci1.wait()
        ci2.wait()
        plsc.subcore_barrier()
        b2vec = b2vv[...]
        w2k = [w2v[kk, :] for kk in range(KBLK)]
        lane_iota = lax.broadcasted_iota(jnp.int32, (LANES,), 0)
        last_mask = lane_iota == (LANES - 1)
        rot_idx = [(lane_iota + r) & (LANES - 1) for r in (8, 4, 2, 1)]

        def issue(c, buf):
            off = c * CHUNK
            pltpu.async_copy(
                aspm.at[sidx.at[pl.ds(off, CHUNK)]], arows.at[buf], sems_a[buf]
            )
            pltpu.async_copy(
                bspm.at[didx.at[pl.ds(off, CHUNK)]], brows.at[buf], sems_b[buf]
            )

        def wait_rows(buf):
            pltpu.make_async_copy(
                aspm.at[sidx.at[pl.ds(0, CHUNK)]], arows.at[buf], sems_a[buf]
            ).wait()
            pltpu.make_async_copy(
                bspm.at[didx.at[pl.ds(0, CHUNK)]], brows.at[buf], sems_b[buf]
            ).wait()

        def drain_out(buf):
            pltpu.make_async_copy(
                outv.at[buf], out_hbm.at[pl.ds(0, CHUNK)], sems_o[buf]
            ).wait()

        def compute(c, buf):
            ar = arows.at[buf]
            br = brows.at[buf]
            ov = outv.at[buf]

            @pl.when(c >= 2)
            def _():
                drain_out(buf)

            @plsc.parallel_loop(0, CHUNK, unroll=4)
            def edge_body(e):
                prods = []
                for kk in range(KBLK):
                    sl = pl.ds(kk * LANES, LANES)
                    va = plsc.bitcast(ar[e, sl], jnp.bfloat16)
                    vb = plsc.bitcast(br[e, sl], jnp.bfloat16)
                    r = jnp.maximum(va + vb, jnp.bfloat16(0.0))
                    prods.append(r * w2k[kk])
                t = (prods[0] + prods[1]) + (prods[2] + prods[3])
                ti = plsc.bitcast(t, jnp.int32)
                lo = plsc.bitcast(ti << 16, jnp.float32)
                hi = plsc.bitcast(ti & jnp.int32(-65536), jnp.float32)
                tot = lo + hi
                for ridx in rot_idx:
                    tot = tot + tot[ridx]
                plsc.store_scatter(
                    ov,
                    [jnp.full((LANES,), 0, jnp.int32) + e],
                    tot,
                    mask=last_mask,
                )

            for j in range(CHUNK // LANES):
                sl = pl.ds(j * LANES, LANES)
                ov[sl] = 1.0 / (1.0 + jnp.exp(-(ov[sl] + b2vec)))
            pltpu.async_copy(
                ov, out_hbm.at[pl.ds(base0 + c * CHUNK, CHUNK)], sems_o[buf]
            )

        issue(0, 0)

        def pair_body(p, carry):
            c0 = 2 * p
            wait_rows(0)
            issue(c0 + 1, 1)
            compute(c0, 0)
            wait_rows(1)
            issue(c0 + 2, 0)
            compute(c0 + 1, 1)
            return carry

        lax.fori_loop(0, N_PAIRS, pair_body, 0)
        wait_rows(0)
        compute(jnp.int32(N_CHUNKS - 1), 0)
        drain_out(0)
        drain_out(1)

    return k(a_tab, b_tab, edge_index, w2r, b2v)


def kernel(x, edge_index, W1, b1, W2, b2):
    w1a = W1[:FEAT]
    w1b = W1[FEAT:]
    b1r = b1.reshape(1, FEAT)
    a_tab, b_tab = _precompute_tables(x, w1a, w1b, b1r)
    w2f = W2.reshape(FEAT)
    w2r = jnp.stack(
        [w2f[: FEAT // 2].reshape(KBLK, LANES), w2f[FEAT // 2 :].reshape(KBLK, LANES)],
        axis=-1,
    ).reshape(KBLK, 2 * LANES).astype(jnp.bfloat16)
    b2v = jnp.broadcast_to(b2, (LANES,))
    return _edge_scores(a_tab, b_tab, edge_index, w2r, b2v)
